# baseline (speedup 1.0000x reference)
import functools

import jax
import jax.numpy as jnp
from jax import lax
from jax.experimental import pallas as pl
from jax.experimental.pallas import tpu as pltpu
from jax.experimental.pallas import tpu_sc as plsc

_L = 16
_NSLOT = 3


@functools.lru_cache(maxsize=None)
def _build(B: int, W: int, H: int):
    NW = 32
    WC = 8
    units = (B * W) // WC
    assert units * WC == B * W and W % WC == 0
    upb = W // WC
    HG = H // _L
    assert HG * _L == H
    MAXK = (units + NW - 1) // NW
    full_k = units - (MAXK - 1) * NW

    mesh = plsc.VectorSubcoreMesh(core_axis_name="c", subcore_axis_name="s")

    slot_bufs = []
    for _ in range(_NSLOT):
        slot_bufs += [
            pltpu.VMEM((3, WC, H), jnp.float32),
            pltpu.VMEM((WC, 4, H), jnp.float32),
            pltpu.VMEM((WC, 4, H), jnp.float32),
        ]
    sems = [pltpu.SemaphoreType.DMA for _ in range(2 * _NSLOT)]

    @functools.partial(
        pl.kernel,
        mesh=mesh,
        out_type=(
            jax.ShapeDtypeStruct((B, W, 4, H), jnp.float32),
            jax.ShapeDtypeStruct((B, W, 4, H), jnp.float32),
        ),
        scratch_types=slot_bufs + sems,
        compiler_params=pltpu.CompilerParams(needs_layout_passes=False),
    )
    def body(in_hbm, bel_hbm, pl_hbm, *sc):
        bufs = [sc[3 * s:3 * s + 3] for s in range(_NSLOT)]
        sin = sc[3 * _NSLOT:4 * _NSLOT]
        sout = sc[4 * _NSLOT:]
        wid = lax.axis_index("s") * 2 + lax.axis_index("c")
        zero_v = jnp.zeros((_L,), jnp.float32)
        one_v = jnp.ones((_L,), jnp.float32)

        def const_fill(w, c):
            for _, bels, pls_ in bufs:
                for hg in range(HG):
                    hs = pl.ds(hg * _L, _L)
                    bels[w, 0, hs] = zero_v
                    bels[w, 3, hs] = one_v
                    pls_[w, 0, hs] = zero_v
                    pls_[w, 3, hs] = one_v
            return c

        lax.fori_loop(0, WC, const_fill, 0)

        def unit_pos(k):
            u = k * NW + wid
            return u // upb, pl.ds((u % upb) * WC, WC)

        def in_copies(k):
            b, ws = unit_pos(k)
            xbuf = bufs[k % _NSLOT][0]
            return [(in_hbm.at[b, pl.ds(0, 3), ws], xbuf, sin[k % _NSLOT])]

        def out_copies(k):
            b, ws = unit_pos(k)
            _, bels, pls_ = bufs[k % _NSLOT]
            sem = sout[k % _NSLOT]
            return [(bels, bel_hbm.at[b, ws], sem),
                    (pls_, pl_hbm.at[b, ws], sem)]

        def issue(copies):
            for src, dst, sem in copies:
                pltpu.async_copy(src, dst, sem)

        def drain(copies):
            for src, dst, sem in copies:
                pltpu.make_async_copy(src, dst, sem).wait()

        def compute(k):
            xbuf, bels, pls_ = bufs[k % _NSLOT]

            def wbody(w, c):
                for hg in range(HG):
                    hs = pl.ds(hg * _L, _L)
                    x0 = xbuf[0, w, hs]
                    x1 = xbuf[1, w, hs]
                    x2 = xbuf[2, w, hs]
                    bels[w, 1, hs] = x0
                    bels[w, 2, hs] = x1
                    pls_[w, 1, hs] = x0 + x2
                    pls_[w, 2, hs] = x1 + x2
                return c

            lax.fori_loop(0, WC, wbody, 0)

        def guarded(k, fn):
            if k < MAXK - 1 or full_k == NW:
                fn()
            else:
                pl.when(wid < full_k)(fn)

        issue(in_copies(0))
        for k in range(MAXK):
            if k >= 2:
                drain(out_copies(k - 2))
            if k + 1 < MAXK:
                guarded(k + 1, lambda k=k: issue(in_copies(k + 1)))

            def stage(k=k):
                drain(in_copies(k))
                compute(k)
                issue(out_copies(k))

            guarded(k, stage)
        drain(out_copies(MAXK - 2))
        guarded(MAXK - 1, lambda: drain(out_copies(MAXK - 1)))

    return body


def kernel(inputs):
    B, H, W, C = inputs.shape
    assert C == 3, "kernel specialized for a 2-class frame (3 input channels)"
    xt = jnp.transpose(inputs, (0, 3, 2, 1))
    bel_t, pl_t = _build(B, W, H)(xt)
    bel = jnp.transpose(bel_t, (0, 3, 1, 2))
    pl_full = jnp.transpose(pl_t, (0, 3, 1, 2))
    return (bel, pl_full)


# speedup vs baseline: 1.0565x; 1.0565x over previous
import functools

import jax
import jax.numpy as jnp
from jax import lax
from jax.experimental import pallas as pl
from jax.experimental.pallas import tpu as pltpu
from jax.experimental.pallas import tpu_sc as plsc

_L = 16
_NSLOT = 3


@functools.lru_cache(maxsize=None)
def _build(B: int, W: int, H: int):
    NW = 32
    WC = 16
    units = (B * W) // WC
    assert units * WC == B * W and W % WC == 0
    upb = W // WC
    HG = H // _L
    assert HG * _L == H
    MAXK = (units + NW - 1) // NW
    full_k = units - (MAXK - 1) * NW

    mesh = plsc.VectorSubcoreMesh(core_axis_name="c", subcore_axis_name="s")

    data_bufs = [pltpu.VMEM((WC, H), jnp.float32) for _ in range(5 * _NSLOT)]
    const_bufs = [pltpu.VMEM((WC, H), jnp.float32) for _ in range(2)]
    sems = [pltpu.SemaphoreType.DMA for _ in range(2 * _NSLOT)]

    @functools.partial(
        pl.kernel,
        mesh=mesh,
        out_type=(
            jax.ShapeDtypeStruct((B, W, 4, H), jnp.float32),
            jax.ShapeDtypeStruct((B, W, 4, H), jnp.float32),
        ),
        scratch_types=data_bufs + const_bufs + sems,
        compiler_params=pltpu.CompilerParams(needs_layout_passes=False),
    )
    def body(in_hbm, bel_hbm, pl_hbm, *sc):
        bufs = [sc[5 * s:5 * s + 5] for s in range(_NSLOT)]
        zb, ob = sc[5 * _NSLOT], sc[5 * _NSLOT + 1]
        sin = sc[5 * _NSLOT + 2:5 * _NSLOT + 2 + _NSLOT]
        sout = sc[5 * _NSLOT + 2 + _NSLOT:]
        wid = lax.axis_index("s") * 2 + lax.axis_index("c")
        zero_v = jnp.zeros((_L,), jnp.float32)
        one_v = jnp.ones((_L,), jnp.float32)

        def const_fill(w, c):
            for hg in range(HG):
                hs = pl.ds(hg * _L, _L)
                zb[w, hs] = zero_v
                ob[w, hs] = one_v
            return c

        lax.fori_loop(0, WC, const_fill, 0)

        def unit_pos(k):
            u = k * NW + wid
            return u // upb, pl.ds((u % upb) * WC, WC)

        def in_copies(k):
            b, ws = unit_pos(k)
            x0s, x1s, x2s, _, _ = bufs[k % _NSLOT]
            sem = sin[k % _NSLOT]
            return [(in_hbm.at[b, 0, ws], x0s, sem),
                    (in_hbm.at[b, 1, ws], x1s, sem),
                    (in_hbm.at[b, 2, ws], x2s, sem)]

        def out_copies(k):
            b, ws = unit_pos(k)
            x0s, x1s, x2s, p0s, p1s = bufs[k % _NSLOT]
            sem = sout[k % _NSLOT]
            return [(zb, bel_hbm.at[b, ws, 0], sem),
                    (x0s, bel_hbm.at[b, ws, 1], sem),
                    (x1s, bel_hbm.at[b, ws, 2], sem),
                    (ob, bel_hbm.at[b, ws, 3], sem),
                    (zb, pl_hbm.at[b, ws, 0], sem),
                    (p0s, pl_hbm.at[b, ws, 1], sem),
                    (p1s, pl_hbm.at[b, ws, 2], sem),
                    (ob, pl_hbm.at[b, ws, 3], sem)]

        def issue(copies):
            for src, dst, sem in copies:
                pltpu.async_copy(src, dst, sem)

        def drain(copies):
            for src, dst, sem in copies:
                pltpu.make_async_copy(src, dst, sem).wait()

        def compute(k):
            x0s, x1s, x2s, p0s, p1s = bufs[k % _NSLOT]

            def wbody(w, c):
                for hg in range(HG):
                    hs = pl.ds(hg * _L, _L)
                    v2 = x2s[w, hs]
                    p0s[w, hs] = x0s[w, hs] + v2
                    p1s[w, hs] = x1s[w, hs] + v2
                return c

            lax.fori_loop(0, WC, wbody, 0)

        def guarded(k, fn):
            if k < MAXK - 1 or full_k == NW:
                fn()
            else:
                pl.when(wid < full_k)(fn)

        issue(in_copies(0))
        for k in range(MAXK):
            if k >= 2:
                drain(out_copies(k - 2))
            if k + 1 < MAXK:
                guarded(k + 1, lambda k=k: issue(in_copies(k + 1)))

            def stage(k=k):
                drain(in_copies(k))
                compute(k)
                issue(out_copies(k))

            guarded(k, stage)
        drain(out_copies(MAXK - 2))
        guarded(MAXK - 1, lambda: drain(out_copies(MAXK - 1)))

    return body


def kernel(inputs):
    B, H, W, C = inputs.shape
    assert C == 3, "kernel specialized for a 2-class frame (3 input channels)"
    xt = jnp.transpose(inputs, (0, 3, 2, 1))
    bel_t, pl_t = _build(B, W, H)(xt)
    bel = jnp.transpose(bel_t, (0, 3, 1, 2))
    pl_full = jnp.transpose(pl_t, (0, 3, 1, 2))
    return (bel, pl_full)


# speedup vs baseline: 1.0577x; 1.0011x over previous
import functools

import jax
import jax.numpy as jnp
from jax import lax
from jax.experimental import pallas as pl
from jax.experimental.pallas import tpu as pltpu
from jax.experimental.pallas import tpu_sc as plsc

_L = 16
_NSLOT = 3


@functools.lru_cache(maxsize=None)
def _build(B: int, W: int, H: int):
    NW = 32
    WC = 16
    units = (B * W) // WC
    assert units * WC == B * W and W % WC == 0
    upb = W // WC
    HG = H // _L
    assert HG * _L == H
    MAXK = (units + NW - 1) // NW
    full_k = units - (MAXK - 1) * NW

    mesh = plsc.VectorSubcoreMesh(core_axis_name="c", subcore_axis_name="s")

    data_bufs = [pltpu.VMEM((WC, H), jnp.float32) for _ in range(5 * _NSLOT)]
    const_bufs = [pltpu.VMEM((WC, H), jnp.float32) for _ in range(2)]
    sems = [pltpu.SemaphoreType.DMA for _ in range(2 * _NSLOT)]

    @functools.partial(
        pl.kernel,
        mesh=mesh,
        out_type=(
            jax.ShapeDtypeStruct((B, W, 4, H), jnp.float32),
            jax.ShapeDtypeStruct((B, W, 4, H), jnp.float32),
        ),
        scratch_types=data_bufs + const_bufs + sems,
        compiler_params=pltpu.CompilerParams(
            needs_layout_passes=False,
            disable_bounds_checks=True,
            disable_semaphore_checks=True,
            skip_device_barrier=True,
        ),
    )
    def body(in_hbm, bel_hbm, pl_hbm, *sc):
        bufs = [sc[5 * s:5 * s + 5] for s in range(_NSLOT)]
        zb, ob = sc[5 * _NSLOT], sc[5 * _NSLOT + 1]
        sin = sc[5 * _NSLOT + 2:5 * _NSLOT + 2 + _NSLOT]
        sout = sc[5 * _NSLOT + 2 + _NSLOT:]
        wid = lax.axis_index("s") * 2 + lax.axis_index("c")
        zero_v = jnp.zeros((_L,), jnp.float32)
        one_v = jnp.ones((_L,), jnp.float32)

        def const_fill(w, c):
            for hg in range(HG):
                hs = pl.ds(hg * _L, _L)
                zb[w, hs] = zero_v
                ob[w, hs] = one_v
            return c

        lax.fori_loop(0, WC, const_fill, 0)

        def unit_pos(k):
            u = k * NW + wid
            return u // upb, pl.ds((u % upb) * WC, WC)

        def in_copies(k):
            b, ws = unit_pos(k)
            x0s, x1s, x2s, _, _ = bufs[k % _NSLOT]
            sem = sin[k % _NSLOT]
            return [(in_hbm.at[b, 0, ws], x0s, sem),
                    (in_hbm.at[b, 1, ws], x1s, sem),
                    (in_hbm.at[b, 2, ws], x2s, sem)]

        def out_copies(k):
            b, ws = unit_pos(k)
            x0s, x1s, x2s, p0s, p1s = bufs[k % _NSLOT]
            sem = sout[k % _NSLOT]
            return [(zb, bel_hbm.at[b, ws, 0], sem),
                    (x0s, bel_hbm.at[b, ws, 1], sem),
                    (x1s, bel_hbm.at[b, ws, 2], sem),
                    (ob, bel_hbm.at[b, ws, 3], sem),
                    (zb, pl_hbm.at[b, ws, 0], sem),
                    (p0s, pl_hbm.at[b, ws, 1], sem),
                    (p1s, pl_hbm.at[b, ws, 2], sem),
                    (ob, pl_hbm.at[b, ws, 3], sem)]

        def issue(copies):
            for src, dst, sem in copies:
                pltpu.async_copy(src, dst, sem)

        def drain(copies):
            for src, dst, sem in copies:
                pltpu.make_async_copy(src, dst, sem).wait()

        def compute(k):
            x0s, x1s, x2s, p0s, p1s = bufs[k % _NSLOT]

            def wbody(w, c):
                for hg in range(HG):
                    hs = pl.ds(hg * _L, _L)
                    v2 = x2s[w, hs]
                    p0s[w, hs] = x0s[w, hs] + v2
                    p1s[w, hs] = x1s[w, hs] + v2
                return c

            lax.fori_loop(0, WC, wbody, 0)

        def guarded(k, fn):
            if k < MAXK - 1 or full_k == NW:
                fn()
            else:
                pl.when(wid < full_k)(fn)

        issue(in_copies(0))
        for k in range(MAXK):
            if k >= 2:
                drain(out_copies(k - 2))
            if k + 1 < MAXK:
                guarded(k + 1, lambda k=k: issue(in_copies(k + 1)))

            def stage(k=k):
                drain(in_copies(k))
                compute(k)
                issue(out_copies(k))

            guarded(k, stage)
        drain(out_copies(MAXK - 2))
        guarded(MAXK - 1, lambda: drain(out_copies(MAXK - 1)))

    return body


def kernel(inputs):
    B, H, W, C = inputs.shape
    assert C == 3, "kernel specialized for a 2-class frame (3 input channels)"
    xt = jnp.transpose(inputs, (0, 3, 2, 1))
    bel_t, pl_t = _build(B, W, H)(xt)
    bel = jnp.transpose(bel_t, (0, 3, 1, 2))
    pl_full = jnp.transpose(pl_t, (0, 3, 1, 2))
    return (bel, pl_full)


# speedup vs baseline: 1.0967x; 1.0369x over previous
import functools

import jax
import jax.numpy as jnp
from jax import lax
from jax.experimental import pallas as pl
from jax.experimental.pallas import tpu as pltpu
from jax.experimental.pallas import tpu_sc as plsc

_L = 16
_NSLOT = 3


@functools.lru_cache(maxsize=None)
def _build(B: int, W: int, H: int):
    NW = 32
    WC = 16
    units = (B * W) // WC
    assert units * WC == B * W and W % WC == 0
    upb = W // WC
    HG = H // _L
    assert HG * _L == H
    MAXK = (units + NW - 1) // NW
    full_k = units - (MAXK - 1) * NW

    mesh = plsc.VectorSubcoreMesh(core_axis_name="c", subcore_axis_name="s")

    data_bufs = [pltpu.VMEM((WC, H), jnp.float32) for _ in range(5 * _NSLOT)]
    const_bufs = [pltpu.VMEM((WC, H), jnp.float32) for _ in range(2)]
    sems = [pltpu.SemaphoreType.DMA for _ in range(2 * _NSLOT)]

    @functools.partial(
        pl.kernel,
        mesh=mesh,
        out_type=(
            jax.ShapeDtypeStruct((B, W, 4, H), jnp.float32),
            jax.ShapeDtypeStruct((B, W, 4, H), jnp.float32),
        ),
        scratch_types=data_bufs + const_bufs + sems,
        compiler_params=pltpu.CompilerParams(
            needs_layout_passes=False,
            disable_bounds_checks=True,
            disable_semaphore_checks=True,
            skip_device_barrier=True,
        ),
    )
    def body(in_hbm, bel_hbm, pl_hbm, *sc):
        bufs = [sc[5 * s:5 * s + 5] for s in range(_NSLOT)]
        zb, ob = sc[5 * _NSLOT], sc[5 * _NSLOT + 1]
        sin = sc[5 * _NSLOT + 2:5 * _NSLOT + 2 + _NSLOT]
        sout = sc[5 * _NSLOT + 2 + _NSLOT:]
        wid = lax.axis_index("s") * 2 + lax.axis_index("c")
        zero_v = jnp.zeros((_L,), jnp.float32)
        one_v = jnp.ones((_L,), jnp.float32)

        def const_fill(w, c):
            def hbody(hg, c2):
                hs = pl.ds(hg * _L, _L)
                zb[w, hs] = zero_v
                ob[w, hs] = one_v
                return c2

            lax.fori_loop(0, HG, hbody, 0)
            return c

        lax.fori_loop(0, WC, const_fill, 0)

        def unit_pos(k):
            u = k * NW + wid
            return u // upb, pl.ds((u % upb) * WC, WC)

        def in_copies(k):
            b, ws = unit_pos(k)
            x0s, x1s, x2s, _, _ = bufs[k % _NSLOT]
            sem = sin[k % _NSLOT]
            return [(in_hbm.at[b, 0, ws], x0s, sem),
                    (in_hbm.at[b, 1, ws], x1s, sem),
                    (in_hbm.at[b, 2, ws], x2s, sem)]

        def out_copies(k):
            b, ws = unit_pos(k)
            x0s, x1s, x2s, p0s, p1s = bufs[k % _NSLOT]
            sem = sout[k % _NSLOT]
            return [(zb, bel_hbm.at[b, ws, 0], sem),
                    (x0s, bel_hbm.at[b, ws, 1], sem),
                    (x1s, bel_hbm.at[b, ws, 2], sem),
                    (ob, bel_hbm.at[b, ws, 3], sem),
                    (zb, pl_hbm.at[b, ws, 0], sem),
                    (p0s, pl_hbm.at[b, ws, 1], sem),
                    (p1s, pl_hbm.at[b, ws, 2], sem),
                    (ob, pl_hbm.at[b, ws, 3], sem)]

        def issue(copies):
            for src, dst, sem in copies:
                pltpu.async_copy(src, dst, sem)

        def drain(copies):
            for src, dst, sem in copies:
                pltpu.make_async_copy(src, dst, sem).wait()

        def compute(k):
            x0s, x1s, x2s, p0s, p1s = bufs[k % _NSLOT]

            def wbody(w, c):
                def hbody(hg, c2):
                    hs = pl.ds(hg * _L, _L)
                    v2 = x2s[w, hs]
                    p0s[w, hs] = x0s[w, hs] + v2
                    p1s[w, hs] = x1s[w, hs] + v2
                    return c2

                lax.fori_loop(0, HG, hbody, 0)
                return c

            lax.fori_loop(0, WC, wbody, 0)

        def guarded(k, fn):
            if k < MAXK - 1 or full_k == NW:
                fn()
            else:
                pl.when(wid < full_k)(fn)

        issue(in_copies(0))
        for k in range(MAXK):
            if k >= 2:
                drain(out_copies(k - 2))
            if k + 1 < MAXK:
                guarded(k + 1, lambda k=k: issue(in_copies(k + 1)))

            def stage(k=k):
                drain(in_copies(k))
                compute(k)
                issue(out_copies(k))

            guarded(k, stage)
        drain(out_copies(MAXK - 2))
        guarded(MAXK - 1, lambda: drain(out_copies(MAXK - 1)))

    return body


def kernel(inputs):
    B, H, W, C = inputs.shape
    assert C == 3, "kernel specialized for a 2-class frame (3 input channels)"
    xt = jnp.transpose(inputs, (0, 3, 2, 1))
    bel_t, pl_t = _build(B, W, H)(xt)
    bel = jnp.transpose(bel_t, (0, 3, 1, 2))
    pl_full = jnp.transpose(pl_t, (0, 3, 1, 2))
    return (bel, pl_full)


# speedup vs baseline: 1.1140x; 1.0158x over previous
import functools

import jax
import jax.numpy as jnp
from jax import lax
from jax.experimental import pallas as pl
from jax.experimental.pallas import tpu as pltpu
from jax.experimental.pallas import tpu_sc as plsc

_L = 16
_NSLOT = 3


@functools.lru_cache(maxsize=None)
def _build(B: int, W: int, H: int):
    NW = 32
    WC = 16
    units = (B * W) // WC
    assert units * WC == B * W and W % WC == 0
    upb = W // WC
    HG = H // _L
    assert HG * _L == H
    MAXK = (units + NW - 1) // NW
    full_k = units - (MAXK - 1) * NW

    mesh = plsc.VectorSubcoreMesh(core_axis_name="c", subcore_axis_name="s")

    data_bufs = [pltpu.VMEM((WC, H), jnp.float32) for _ in range(5 * _NSLOT)]
    const_bufs = [pltpu.VMEM((WC, H), jnp.float32) for _ in range(2)]
    sems = [pltpu.SemaphoreType.DMA for _ in range(2 * _NSLOT)]

    @functools.partial(
        pl.kernel,
        mesh=mesh,
        out_type=(
            jax.ShapeDtypeStruct((B, W, 4, H), jnp.float32),
            jax.ShapeDtypeStruct((B, W, 4, H), jnp.float32),
        ),
        scratch_types=data_bufs + const_bufs + sems,
        compiler_params=pltpu.CompilerParams(
            needs_layout_passes=False,
            disable_bounds_checks=True,
            disable_semaphore_checks=True,
            skip_device_barrier=True,
        ),
    )
    def body(in_hbm, bel_hbm, pl_hbm, *sc):
        bufs = [sc[5 * s:5 * s + 5] for s in range(_NSLOT)]
        zb, ob = sc[5 * _NSLOT], sc[5 * _NSLOT + 1]
        sin = sc[5 * _NSLOT + 2:5 * _NSLOT + 2 + _NSLOT]
        sout = sc[5 * _NSLOT + 2 + _NSLOT:]
        wid = lax.axis_index("s") * 2 + lax.axis_index("c")
        zero_v = jnp.zeros((_L,), jnp.float32)
        one_v = jnp.ones((_L,), jnp.float32)

        def const_fill(w, c):
            def hbody(hg, c2):
                hs = pl.ds(hg * _L, _L)
                zb[w, hs] = zero_v
                ob[w, hs] = one_v
                return c2

            lax.fori_loop(0, HG, hbody, 0)
            return c

        lax.fori_loop(0, WC, const_fill, 0)

        def unit_pos(k):
            u = k * NW + wid
            return u // upb, pl.ds((u % upb) * WC, WC)

        def in_copies(k, s):
            b, ws = unit_pos(k)
            x0s, x1s, x2s, _, _ = bufs[s]
            sem = sin[s]
            return [(in_hbm.at[b, 0, ws], x0s, sem),
                    (in_hbm.at[b, 1, ws], x1s, sem),
                    (in_hbm.at[b, 2, ws], x2s, sem)]

        def out_copies(k, s):
            b, ws = unit_pos(k)
            x0s, x1s, x2s, p0s, p1s = bufs[s]
            sem = sout[s]
            return [(zb, bel_hbm.at[b, ws, 0], sem),
                    (x0s, bel_hbm.at[b, ws, 1], sem),
                    (x1s, bel_hbm.at[b, ws, 2], sem),
                    (ob, bel_hbm.at[b, ws, 3], sem),
                    (zb, pl_hbm.at[b, ws, 0], sem),
                    (p0s, pl_hbm.at[b, ws, 1], sem),
                    (p1s, pl_hbm.at[b, ws, 2], sem),
                    (ob, pl_hbm.at[b, ws, 3], sem)]

        def issue(copies):
            for src, dst, sem in copies:
                pltpu.async_copy(src, dst, sem)

        def drain(copies):
            for src, dst, sem in copies:
                pltpu.make_async_copy(src, dst, sem).wait()

        def compute(s):
            x0s, x1s, x2s, p0s, p1s = bufs[s]

            def wbody(w, c):
                def hbody(hg, c2):
                    hs = pl.ds(hg * _L, _L)
                    v2 = x2s[w, hs]
                    p0s[w, hs] = x0s[w, hs] + v2
                    p1s[w, hs] = x1s[w, hs] + v2
                    return c2

                lax.fori_loop(0, HG, hbody, 0)
                return c

            lax.fori_loop(0, WC, wbody, 0)

        def guarded(cond, fn):
            if cond is True:
                fn()
            else:
                pl.when(cond)(fn)

        def stage_body(k, s):
            def body():
                drain(in_copies(k, s))
                compute(s)
                issue(out_copies(k, s))
            return body

        def stage(k, s, first=False):
            if not first:
                drain(out_copies(k - 2, (s - 2) % _NSLOT))
            issue(in_copies(k + 1, (s + 1) % _NSLOT))
            stage_body(k, s)()

        issue(in_copies(0, 0))
        stage(0, 0, first=True)
        stage(1, 1, first=True)
        nbody = (MAXK - 4) // _NSLOT
        ktail = 2 + _NSLOT * nbody

        def group(g, c):
            k0 = _NSLOT * g + 2
            for j in range(_NSLOT):
                stage(k0 + j, (2 + j) % _NSLOT)
            return c

        lax.fori_loop(0, nbody, group, 0)
        for k in range(ktail, MAXK):
            s = k % _NSLOT
            if k == MAXK - 1 and full_k != NW:
                drain(out_copies(k - 2, (s - 2) % _NSLOT))
                pl.when(wid < full_k)(stage_body(k, s))
            else:
                nxt = k + 1
                if nxt < MAXK and (nxt < MAXK - 1 or full_k == NW):
                    stage(k, s)
                else:
                    drain(out_copies(k - 2, (s - 2) % _NSLOT))
                    if nxt < MAXK and nxt == MAXK - 1 and full_k != NW:
                        def issue_next(nxt=nxt, s=s):
                            issue(in_copies(nxt, (s + 1) % _NSLOT))
                        pl.when(wid < full_k)(issue_next)
                    stage_body(k, s)()
        drain(out_copies(MAXK - 2, (MAXK - 2) % _NSLOT))

        def drain_last():
            drain(out_copies(MAXK - 1, (MAXK - 1) % _NSLOT))

        guarded(True if full_k == NW else (wid < full_k), drain_last)

    return body


def kernel(inputs):
    B, H, W, C = inputs.shape
    assert C == 3, "kernel specialized for a 2-class frame (3 input channels)"
    xt = jnp.transpose(inputs, (0, 3, 2, 1))
    bel_t, pl_t = _build(B, W, H)(xt)
    bel = jnp.transpose(bel_t, (0, 3, 1, 2))
    pl_full = jnp.transpose(pl_t, (0, 3, 1, 2))
    return (bel, pl_full)
